# baseline (speedup 1.0000x reference)
import functools

import jax
import jax.numpy as jnp
from jax import lax
from jax.experimental import pallas as pl
from jax.experimental.pallas import tpu as pltpu
from jax.experimental.pallas import tpu_sc as plsc

_K = 200
_BIG = 1e30


def _tc_stage(means_ref, gtT_ref, d_ref, thr_ref, istar_ref, idx0_ref,
              cd1_ref, acc_ref, *, B, N, M):
    b = pl.program_id(0)

    @pl.when(b == 0)
    def _init():
        acc_ref[0] = 0.0
        acc_ref[1] = 0.0

    D = jnp.zeros((N, M), jnp.float32)
    for c in range(3):
        mcol = means_ref[0, :, c:c + 1]
        grow = gtT_ref[0, c:c + 1, :]
        diff = mcol - grow
        D = D + diff * diff
    d_ref[...] = D

    iota = lax.broadcasted_iota(jnp.int32, (N, M), 1)
    rowmin = jnp.min(D, axis=1, keepdims=True)
    e1_sum = jnp.sum(jnp.sqrt(jnp.maximum(rowmin, 1e-9)))
    colmin = jnp.min(D, axis=0, keepdims=True)
    e2_sum = jnp.sum(jnp.sqrt(jnp.maximum(colmin, 1e-9)))
    idx0 = jnp.min(jnp.where(D == rowmin, iota, M), axis=1, keepdims=True)

    bits = lax.bitcast_convert_type(D, jnp.int32)
    lo = lax.bitcast_convert_type(rowmin, jnp.int32) - 1
    hi = lax.bitcast_convert_type(
        jnp.max(D, axis=1, keepdims=True), jnp.int32)

    def vcond(carry):
        lo, hi = carry
        return jnp.max(hi - lo) > 1

    def vbody(carry):
        lo, hi = carry
        mid = lo + (hi - lo) // 2
        cnt = jnp.sum((bits <= mid).astype(jnp.int32), axis=1, keepdims=True)
        ge = cnt >= (_K + 1)
        return jnp.where(ge, lo, mid), jnp.where(ge, mid, hi)

    lo, hi = lax.while_loop(vcond, vbody, (lo, hi))
    thr = hi

    eq = bits == thr
    n_eq = jnp.sum(eq.astype(jnp.int32))

    def no_ties():
        return jnp.full((N, 1), M - 1, jnp.int32)

    def with_ties():
        cnt_lt = jnp.sum((bits < thr).astype(jnp.int32), axis=1,
                         keepdims=True)
        need = (_K + 1) - cnt_lt
        lo2 = jnp.full((N, 1), -1, jnp.int32)
        hi2 = jnp.full((N, 1), M - 1, jnp.int32)

        def ibody(_, carry):
            lo, hi = carry
            mid = lo + (hi - lo) // 2
            cnt = jnp.sum((eq & (iota <= mid)).astype(jnp.int32), axis=1,
                          keepdims=True)
            ge = cnt >= need
            return jnp.where(ge, lo, mid), jnp.where(ge, mid, hi)

        lo2, hi2 = lax.fori_loop(0, 13, ibody, (lo2, hi2))
        return hi2

    istar = lax.cond(n_eq == N, no_ties, with_ties)

    thr_ref[...] = lax.bitcast_convert_type(thr, jnp.float32).reshape(1, 1, N)
    istar_ref[...] = istar.reshape(1, 1, N)
    idx0_ref[...] = idx0.reshape(1, 1, N)

    acc_ref[0] = acc_ref[0] + e1_sum
    acc_ref[1] = acc_ref[1] + e2_sum

    @pl.when(b == B - 1)
    def _finish():
        cd1 = (acc_ref[0] / (B * N) + acc_ref[1] / (B * M)) * 0.5 * 1000.0
        cd1_ref[...] = jnp.reshape(cd1, (1, 1))


def _tc_call(means, gtT, B, N, M):
    body = functools.partial(_tc_stage, B=B, N=N, M=M)
    return pl.pallas_call(
        body,
        grid=(B,),
        in_specs=[
            pl.BlockSpec((1, N, 3), lambda b: (b, 0, 0)),
            pl.BlockSpec((1, 3, M), lambda b: (b, 0, 0)),
        ],
        out_specs=[
            pl.BlockSpec((N, M), lambda b: (b, 0)),
            pl.BlockSpec((1, 1, N), lambda b: (b, 0, 0)),
            pl.BlockSpec((1, 1, N), lambda b: (b, 0, 0)),
            pl.BlockSpec((1, 1, N), lambda b: (b, 0, 0)),
            pl.BlockSpec((1, 1), lambda b: (0, 0)),
        ],
        out_shape=[
            jax.ShapeDtypeStruct((B * N, M), jnp.float32),
            jax.ShapeDtypeStruct((B, 1, N), jnp.float32),
            jax.ShapeDtypeStruct((B, 1, N), jnp.int32),
            jax.ShapeDtypeStruct((B, 1, N), jnp.int32),
            jax.ShapeDtypeStruct((1, 1), jnp.float32),
        ],
        scratch_shapes=[pltpu.SMEM((2,), jnp.float32)],
        compiler_params=pltpu.CompilerParams(
            dimension_semantics=("arbitrary",)),
    )(means, gtT)


def _sqrt16(x):
    b = lax.bitcast_convert_type(x, jnp.int32)
    y = lax.bitcast_convert_type((b >> 1) + 0x1FBD1DF5, jnp.float32)
    for _ in range(3):
        y = 0.5 * (y + x / y)
    return y


def _sc_call(dflat, thr, istar, idx0, spflat, gtflat, B, N, S, M):
    NC, NS, L = 2, 16, 16
    NW = NC * NS
    KPW = (B * N) // NW
    NCH = M // L
    KCH = (_K + L - 1) // L
    KPAD = KCH * L

    mesh = plsc.VectorSubcoreMesh(core_axis_name="c", subcore_axis_name="s")

    @functools.partial(
        pl.kernel, mesh=mesh,
        compiler_params=pltpu.CompilerParams(needs_layout_passes=False),
        out_type=jax.ShapeDtypeStruct((NW * L,), jnp.float32),
        scratch_types=[
            pltpu.VMEM((M,), jnp.float32),
            pltpu.VMEM((M,), jnp.float32),
            pltpu.VMEM((M,), jnp.float32),
            pltpu.VMEM((M,), jnp.float32),
            pltpu.VMEM((M,), jnp.float32),
            pltpu.VMEM((KPAD + L,), jnp.int32),
            pltpu.VMEM((KPAD,), jnp.float32),
            pltpu.VMEM((KPAD,), jnp.float32),
            pltpu.VMEM((KPAD,), jnp.float32),
            pltpu.VMEM((KPAD,), jnp.float32),
            pltpu.VMEM((KPW * S * 3,), jnp.float32),
            pltpu.VMEM((L,), jnp.float32),
            pltpu.VMEM((L,), jnp.int32),
            pltpu.VMEM((L,), jnp.int32),
            pltpu.VMEM((L,), jnp.float32),
            pltpu.SemaphoreType.DMA,
            pltpu.SemaphoreType.DMA,
        ],
    )
    def sc_kernel(d_hbm, thr_hbm, istar_hbm, idx0_hbm, sp_hbm, gt_hbm,
                  out_hbm, drow0, drow1, gx, gy, gz, cand, cgx, cgy, cgz,
                  minS, spv, thrv, istarv, idx0v, outbuf, sem0, sem1):
        wid = lax.axis_index("s") * NC + lax.axis_index("c")
        batch = (wid * KPW) // N
        lanes = lax.iota(jnp.int32, L)

        pltpu.sync_copy(gt_hbm.at[pl.ds((batch * 3 + 0) * M, M)], gx)
        pltpu.sync_copy(gt_hbm.at[pl.ds((batch * 3 + 1) * M, M)], gy)
        pltpu.sync_copy(gt_hbm.at[pl.ds((batch * 3 + 2) * M, M)], gz)
        pltpu.sync_copy(thr_hbm.at[pl.ds(wid * KPW, L)], thrv)
        pltpu.sync_copy(istar_hbm.at[pl.ds(wid * KPW, L)], istarv)
        pltpu.sync_copy(idx0_hbm.at[pl.ds(wid * KPW, L)], idx0v)
        pltpu.sync_copy(sp_hbm.at[pl.ds(wid * KPW * S * 3, KPW * S * 3)], spv)
        base = wid * KPW
        pltpu.async_copy(d_hbm.at[pl.ds(base * M, M)], drow0, sem0)

        for ch in range(KCH + 1):
            cand[pl.ds(ch * L, L)] = jnp.zeros((L,), jnp.int32)

        def _extract_f(vref, i):
            v = vref[...]
            return lax.reduce_max(jnp.where(lanes == i, v, -_BIG), axes=(0,))

        def _extract_i(vref, i):
            v = vref[...]
            return lax.reduce_max(
                jnp.where(lanes == i, v, -(2 ** 31 - 1)), axes=(0,))

        def per_kp(i, carry, drow):
            s1, s2 = carry
            thr_s = _extract_f(thrv, i)
            istar_s = _extract_i(istarv, i)
            idx0_s = _extract_i(idx0v, i)

            def compact(ch, base):
                dv = drow[pl.ds(ch * L, L)]
                idxv = lanes + ch * L
                m = (dv < thr_s) | ((dv == thr_s) & (idxv <= istar_s))
                m = m & (idxv != idx0_s)
                mi = m.astype(jnp.int32)
                inc = plsc.cumsum(mi)
                rank = (inc - mi) + base
                plsc.store_scatter(cand, [rank], idxv, mask=m)
                return base + inc[L - 1]

            lax.fori_loop(0, NCH, compact, 0)

            def gather(ch, _):
                idxv = cand[pl.ds(ch * L, L)]
                cgx[pl.ds(ch * L, L)] = plsc.load_gather(gx, [idxv])
                cgy[pl.ds(ch * L, L)] = plsc.load_gather(gy, [idxv])
                cgz[pl.ds(ch * L, L)] = plsc.load_gather(gz, [idxv])
                minS[pl.ds(ch * L, L)] = jnp.full((L,), _BIG, jnp.float32)
                return 0

            lax.fori_loop(0, KCH, gather, 0)

            d1a = jnp.full((L,), 1e-9, jnp.float32)
            d1b = jnp.full((L,), 1e-9, jnp.float32)
            spo = i * (S * 3)
            for smp in range(S):
                svec = spv[pl.ds(spo + (smp // L) * L, L)]
                sx = svec[(smp % L)] * 1.0
                svec_y = spv[pl.ds(spo + S + (smp // L) * L, L)]
                sy = svec_y[(smp % L)] * 1.0
                svec_z = spv[pl.ds(spo + 2 * S + (smp // L) * L, L)]
                sz = svec_z[(smp % L)] * 1.0

                def per_chunk(ch, best, sx=sx, sy=sy, sz=sz):
                    valid = (lanes + ch * L) < _K
                    dx = sx - cgx[pl.ds(ch * L, L)]
                    dy = sy - cgy[pl.ds(ch * L, L)]
                    dz = sz - cgz[pl.ds(ch * L, L)]
                    dd = dx * dx + dy * dy + dz * dz
                    dd = jnp.where(valid, dd, _BIG)
                    mold = minS[pl.ds(ch * L, L)]
                    minS[pl.ds(ch * L, L)] = jnp.minimum(mold, dd)
                    return jnp.minimum(best, dd)

                best = lax.fori_loop(0, KCH, per_chunk,
                                     jnp.full((L,), _BIG, jnp.float32))
                bmin = lax.reduce_min(best, axes=(0,))
                if smp < L:
                    d1a = jnp.where(lanes == smp, bmin, d1a)
                else:
                    d1b = jnp.where(lanes == smp - L, bmin, d1b)
            s1 = s1 + lax.reduce_sum(
                _sqrt16(jnp.maximum(d1a, 1e-9)), axes=(0,))
            s1 = s1 + lax.reduce_sum(
                _sqrt16(jnp.maximum(d1b, 1e-9)), axes=(0,))

            def d2sum(ch, acc):
                valid = (lanes + ch * L) < _K
                mv = jnp.maximum(minS[pl.ds(ch * L, L)], 1e-9)
                rv = _sqrt16(mv)
                rv = jnp.where(valid, rv, 0.0)
                return acc + lax.reduce_sum(rv, axes=(0,))

            s2 = lax.fori_loop(0, KCH, d2sum, s2)
            return s1, s2

        def pair(t, carry):
            i0 = t * 2
            kp0 = base + i0
            pltpu.make_async_copy(
                d_hbm.at[pl.ds(kp0 * M, M)], drow0, sem0).wait()
            pltpu.async_copy(d_hbm.at[pl.ds((kp0 + 1) * M, M)], drow1, sem1)
            carry = per_kp(i0, carry, drow0)
            pltpu.make_async_copy(
                d_hbm.at[pl.ds((kp0 + 1) * M, M)], drow1, sem1).wait()

            @pl.when(t < (KPW // 2) - 1)
            def _prefetch():
                pltpu.async_copy(
                    d_hbm.at[pl.ds((kp0 + 2) * M, M)], drow0, sem0)

            return per_kp(i0 + 1, carry, drow1)

        s1, s2 = lax.fori_loop(0, KPW // 2, pair, (0.0, 0.0))

        outbuf[...] = jnp.where(lanes == 0, s1,
                                jnp.where(lanes == 1, s2, 0.0))
        pltpu.sync_copy(outbuf, out_hbm.at[pl.ds(wid * L, L)])

    return sc_kernel(dflat, thr, istar, idx0, spflat, gtflat)


@jax.jit
def kernel(means, sample_points, gt):
    B, N, S, _ = sample_points.shape
    M = gt.shape[1]
    gtT = gt.transpose(0, 2, 1)

    dmat, thr, istar, idx0, cd1 = _tc_call(means, gtT, B, N, M)

    part = _sc_call(dmat.reshape(B * N * M), thr.reshape(B * N),
                    istar.reshape(B * N), idx0.reshape(B * N),
                    sample_points.transpose(0, 1, 3, 2).reshape(B * N * S * 3),
                    gtT.reshape(B * 3 * M), B, N, S, M)
    part = part.reshape(32, 16)
    S1 = jnp.sum(part[:, 0])
    S2 = jnp.sum(part[:, 1])
    cd2 = (S1 / (B * S) + S2 / (B * _K)) * 0.5 * 1000.0
    cd1s = cd1[0, 0]
    return (cd2, cd1s, cd2)


# speedup vs baseline: 1.0046x; 1.0046x over previous
import functools

import jax
import jax.numpy as jnp
from jax import lax
from jax.experimental import pallas as pl
from jax.experimental.pallas import tpu as pltpu
from jax.experimental.pallas import tpu_sc as plsc

_K = 200
_BIG = 1e30


def _tc_stage(means_ref, gtT_ref, d_ref, thr_ref, istar_ref, idx0_ref,
              cd1_ref, acc_ref, *, B, N, M):
    b = pl.program_id(0)

    @pl.when(b == 0)
    def _init():
        acc_ref[0] = 0.0
        acc_ref[1] = 0.0

    D = jnp.zeros((N, M), jnp.float32)
    for c in range(3):
        mcol = means_ref[0, :, c:c + 1]
        grow = gtT_ref[0, c:c + 1, :]
        diff = mcol - grow
        D = D + diff * diff
    d_ref[...] = D

    iota = lax.broadcasted_iota(jnp.int32, (N, M), 1)
    rowmin = jnp.min(D, axis=1, keepdims=True)
    e1_sum = jnp.sum(jnp.sqrt(jnp.maximum(rowmin, 1e-9)))
    colmin = jnp.min(D, axis=0, keepdims=True)
    e2_sum = jnp.sum(jnp.sqrt(jnp.maximum(colmin, 1e-9)))
    idx0 = jnp.min(jnp.where(D == rowmin, iota, M), axis=1, keepdims=True)

    bits = lax.bitcast_convert_type(D, jnp.int32)
    lo = lax.bitcast_convert_type(rowmin, jnp.int32) - 1
    hi = lax.bitcast_convert_type(
        jnp.max(D, axis=1, keepdims=True), jnp.int32)

    def vcond(carry):
        lo, hi = carry
        return jnp.max(hi - lo) > 1

    def vbody(carry):
        lo, hi = carry
        mid = lo + (hi - lo) // 2
        cnt = jnp.sum((bits <= mid).astype(jnp.int32), axis=1, keepdims=True)
        ge = cnt >= (_K + 1)
        return jnp.where(ge, lo, mid), jnp.where(ge, mid, hi)

    lo, hi = lax.while_loop(vcond, vbody, (lo, hi))
    thr = hi

    eq = bits == thr
    n_eq = jnp.sum(eq.astype(jnp.int32))

    def no_ties():
        return jnp.full((N, 1), M - 1, jnp.int32)

    def with_ties():
        cnt_lt = jnp.sum((bits < thr).astype(jnp.int32), axis=1,
                         keepdims=True)
        need = (_K + 1) - cnt_lt
        lo2 = jnp.full((N, 1), -1, jnp.int32)
        hi2 = jnp.full((N, 1), M - 1, jnp.int32)

        def ibody(_, carry):
            lo, hi = carry
            mid = lo + (hi - lo) // 2
            cnt = jnp.sum((eq & (iota <= mid)).astype(jnp.int32), axis=1,
                          keepdims=True)
            ge = cnt >= need
            return jnp.where(ge, lo, mid), jnp.where(ge, mid, hi)

        lo2, hi2 = lax.fori_loop(0, 13, ibody, (lo2, hi2))
        return hi2

    istar = lax.cond(n_eq == N, no_ties, with_ties)

    thr_ref[...] = lax.bitcast_convert_type(thr, jnp.float32).reshape(1, 1, N)
    istar_ref[...] = istar.reshape(1, 1, N)
    idx0_ref[...] = idx0.reshape(1, 1, N)

    acc_ref[0] = acc_ref[0] + e1_sum
    acc_ref[1] = acc_ref[1] + e2_sum

    @pl.when(b == B - 1)
    def _finish():
        cd1 = (acc_ref[0] / (B * N) + acc_ref[1] / (B * M)) * 0.5 * 1000.0
        cd1_ref[...] = jnp.reshape(cd1, (1, 1))


def _tc_call(means, gtT, B, N, M):
    body = functools.partial(_tc_stage, B=B, N=N, M=M)
    return pl.pallas_call(
        body,
        grid=(B,),
        in_specs=[
            pl.BlockSpec((1, N, 3), lambda b: (b, 0, 0)),
            pl.BlockSpec((1, 3, M), lambda b: (b, 0, 0)),
        ],
        out_specs=[
            pl.BlockSpec((N, M), lambda b: (b, 0)),
            pl.BlockSpec((1, 1, N), lambda b: (b, 0, 0)),
            pl.BlockSpec((1, 1, N), lambda b: (b, 0, 0)),
            pl.BlockSpec((1, 1, N), lambda b: (b, 0, 0)),
            pl.BlockSpec((1, 1), lambda b: (0, 0)),
        ],
        out_shape=[
            jax.ShapeDtypeStruct((B * N, M), jnp.float32),
            jax.ShapeDtypeStruct((B, 1, N), jnp.float32),
            jax.ShapeDtypeStruct((B, 1, N), jnp.int32),
            jax.ShapeDtypeStruct((B, 1, N), jnp.int32),
            jax.ShapeDtypeStruct((1, 1), jnp.float32),
        ],
        scratch_shapes=[pltpu.SMEM((2,), jnp.float32)],
        compiler_params=pltpu.CompilerParams(
            dimension_semantics=("arbitrary",)),
    )(means, gtT)


def _sqrt16(x):
    b = lax.bitcast_convert_type(x, jnp.int32)
    y = lax.bitcast_convert_type((b >> 1) + 0x1FBD1DF5, jnp.float32)
    for _ in range(3):
        y = 0.5 * (y + x / y)
    return y


def _sc_call(dflat, thr, istar, idx0, spflat, gtflat, B, N, S, M):
    NC, NS, L = 2, 16, 16
    NW = NC * NS
    KPW = (B * N) // NW
    NCH = M // L
    KCH = (_K + L - 1) // L
    KPAD = KCH * L

    mesh = plsc.VectorSubcoreMesh(core_axis_name="c", subcore_axis_name="s")

    @functools.partial(
        pl.kernel, mesh=mesh,
        compiler_params=pltpu.CompilerParams(needs_layout_passes=False),
        out_type=jax.ShapeDtypeStruct((NW * L,), jnp.float32),
        scratch_types=[
            pltpu.VMEM((M,), jnp.float32),
            pltpu.VMEM((M,), jnp.float32),
            pltpu.VMEM((M,), jnp.float32),
            pltpu.VMEM((M,), jnp.float32),
            pltpu.VMEM((M,), jnp.float32),
            pltpu.VMEM((KPAD + L,), jnp.int32),
            pltpu.VMEM((KPAD,), jnp.float32),
            pltpu.VMEM((KPAD,), jnp.float32),
            pltpu.VMEM((KPAD,), jnp.float32),
            pltpu.VMEM((KPAD,), jnp.float32),
            pltpu.VMEM((KPW * S * 3,), jnp.float32),
            pltpu.VMEM((L,), jnp.float32),
            pltpu.VMEM((L,), jnp.int32),
            pltpu.VMEM((L,), jnp.int32),
            pltpu.VMEM((L,), jnp.float32),
            pltpu.SemaphoreType.DMA,
            pltpu.SemaphoreType.DMA,
        ],
    )
    def sc_kernel(d_hbm, thr_hbm, istar_hbm, idx0_hbm, sp_hbm, gt_hbm,
                  out_hbm, drow0, drow1, gx, gy, gz, cand, cgx, cgy, cgz,
                  minS, spv, thrv, istarv, idx0v, outbuf, sem0, sem1):
        wid = lax.axis_index("s") * NC + lax.axis_index("c")
        batch = (wid * KPW) // N
        lanes = lax.iota(jnp.int32, L)

        pltpu.sync_copy(gt_hbm.at[pl.ds((batch * 3 + 0) * M, M)], gx)
        pltpu.sync_copy(gt_hbm.at[pl.ds((batch * 3 + 1) * M, M)], gy)
        pltpu.sync_copy(gt_hbm.at[pl.ds((batch * 3 + 2) * M, M)], gz)
        pltpu.sync_copy(thr_hbm.at[pl.ds(wid * KPW, L)], thrv)
        pltpu.sync_copy(istar_hbm.at[pl.ds(wid * KPW, L)], istarv)
        pltpu.sync_copy(idx0_hbm.at[pl.ds(wid * KPW, L)], idx0v)
        pltpu.sync_copy(sp_hbm.at[pl.ds(wid * KPW * S * 3, KPW * S * 3)], spv)
        base = wid * KPW
        pltpu.async_copy(d_hbm.at[pl.ds(base * M, M)], drow0, sem0)

        for ch in range(KCH + 1):
            cand[pl.ds(ch * L, L)] = jnp.zeros((L,), jnp.int32)

        def _extract_f(vref, i):
            v = vref[...]
            return lax.reduce_max(jnp.where(lanes == i, v, -_BIG), axes=(0,))

        def _extract_i(vref, i):
            v = vref[...]
            return lax.reduce_max(
                jnp.where(lanes == i, v, -(2 ** 31 - 1)), axes=(0,))

        def per_kp(i, carry, drow):
            s1, s2 = carry
            thr_s = _extract_f(thrv, i)
            istar_s = _extract_i(istarv, i)
            idx0_s = _extract_i(idx0v, i)

            def compact(ch, cnt):
                dv = drow[pl.ds(ch * L, L)]
                idxv = lanes + ch * L
                m = (dv < thr_s) | ((dv == thr_s) & (idxv <= istar_s))
                m = m & (idxv != idx0_s)
                plsc.store_compressed(cand.at[pl.ds(cnt, L)], idxv, mask=m)
                return cnt + lax.reduce_sum(m.astype(jnp.int32), axes=(0,))

            lax.fori_loop(0, NCH, compact, 0, unroll=4)

            def gather(ch, _):
                idxv = cand[pl.ds(ch * L, L)]
                cgx[pl.ds(ch * L, L)] = plsc.load_gather(gx, [idxv])
                cgy[pl.ds(ch * L, L)] = plsc.load_gather(gy, [idxv])
                cgz[pl.ds(ch * L, L)] = plsc.load_gather(gz, [idxv])
                minS[pl.ds(ch * L, L)] = jnp.full((L,), _BIG, jnp.float32)
                return 0

            lax.fori_loop(0, KCH, gather, 0)

            d1a = jnp.full((L,), 1e-9, jnp.float32)
            d1b = jnp.full((L,), 1e-9, jnp.float32)
            spo = i * (S * 3)
            for smp in range(S):
                svec = spv[pl.ds(spo + (smp // L) * L, L)]
                sx = svec[(smp % L)] * 1.0
                svec_y = spv[pl.ds(spo + S + (smp // L) * L, L)]
                sy = svec_y[(smp % L)] * 1.0
                svec_z = spv[pl.ds(spo + 2 * S + (smp // L) * L, L)]
                sz = svec_z[(smp % L)] * 1.0

                def per_chunk(ch, best, sx=sx, sy=sy, sz=sz):
                    valid = (lanes + ch * L) < _K
                    dx = sx - cgx[pl.ds(ch * L, L)]
                    dy = sy - cgy[pl.ds(ch * L, L)]
                    dz = sz - cgz[pl.ds(ch * L, L)]
                    dd = dx * dx + dy * dy + dz * dz
                    dd = jnp.where(valid, dd, _BIG)
                    mold = minS[pl.ds(ch * L, L)]
                    minS[pl.ds(ch * L, L)] = jnp.minimum(mold, dd)
                    return jnp.minimum(best, dd)

                best = lax.fori_loop(0, KCH, per_chunk,
                                     jnp.full((L,), _BIG, jnp.float32),
                                     unroll=2)
                bmin = lax.reduce_min(best, axes=(0,))
                if smp < L:
                    d1a = jnp.where(lanes == smp, bmin, d1a)
                else:
                    d1b = jnp.where(lanes == smp - L, bmin, d1b)
            s1 = s1 + lax.reduce_sum(
                _sqrt16(jnp.maximum(d1a, 1e-9)), axes=(0,))
            s1 = s1 + lax.reduce_sum(
                _sqrt16(jnp.maximum(d1b, 1e-9)), axes=(0,))

            def d2sum(ch, acc):
                valid = (lanes + ch * L) < _K
                mv = jnp.maximum(minS[pl.ds(ch * L, L)], 1e-9)
                rv = _sqrt16(mv)
                rv = jnp.where(valid, rv, 0.0)
                return acc + lax.reduce_sum(rv, axes=(0,))

            s2 = lax.fori_loop(0, KCH, d2sum, s2)
            return s1, s2

        def pair(t, carry):
            i0 = t * 2
            kp0 = base + i0
            pltpu.make_async_copy(
                d_hbm.at[pl.ds(kp0 * M, M)], drow0, sem0).wait()
            pltpu.async_copy(d_hbm.at[pl.ds((kp0 + 1) * M, M)], drow1, sem1)
            carry = per_kp(i0, carry, drow0)
            pltpu.make_async_copy(
                d_hbm.at[pl.ds((kp0 + 1) * M, M)], drow1, sem1).wait()

            @pl.when(t < (KPW // 2) - 1)
            def _prefetch():
                pltpu.async_copy(
                    d_hbm.at[pl.ds((kp0 + 2) * M, M)], drow0, sem0)

            return per_kp(i0 + 1, carry, drow1)

        s1, s2 = lax.fori_loop(0, KPW // 2, pair, (0.0, 0.0))

        outbuf[...] = jnp.where(lanes == 0, s1,
                                jnp.where(lanes == 1, s2, 0.0))
        pltpu.sync_copy(outbuf, out_hbm.at[pl.ds(wid * L, L)])

    return sc_kernel(dflat, thr, istar, idx0, spflat, gtflat)


@jax.jit
def kernel(means, sample_points, gt):
    B, N, S, _ = sample_points.shape
    M = gt.shape[1]
    gtT = gt.transpose(0, 2, 1)

    dmat, thr, istar, idx0, cd1 = _tc_call(means, gtT, B, N, M)

    part = _sc_call(dmat.reshape(B * N * M), thr.reshape(B * N),
                    istar.reshape(B * N), idx0.reshape(B * N),
                    sample_points.transpose(0, 1, 3, 2).reshape(B * N * S * 3),
                    gtT.reshape(B * 3 * M), B, N, S, M)
    part = part.reshape(32, 16)
    S1 = jnp.sum(part[:, 0])
    S2 = jnp.sum(part[:, 1])
    cd2 = (S1 / (B * S) + S2 / (B * _K)) * 0.5 * 1000.0
    cd1s = cd1[0, 0]
    return (cd2, cd1s, cd2)


# speedup vs baseline: 1.1083x; 1.1032x over previous
import functools

import jax
import jax.numpy as jnp
from jax import lax
from jax.experimental import pallas as pl
from jax.experimental.pallas import tpu as pltpu
from jax.experimental.pallas import tpu_sc as plsc

_K = 200
_BIG = 1e30


def _tc_stage(means_ref, gtT_ref, d_ref, thr_ref, istar_ref, idx0_ref,
              cd1_ref, acc_ref, *, B, N, M):
    b = pl.program_id(0)

    @pl.when(b == 0)
    def _init():
        acc_ref[0] = 0.0
        acc_ref[1] = 0.0

    D = jnp.zeros((N, M), jnp.float32)
    for c in range(3):
        mcol = means_ref[0, :, c:c + 1]
        grow = gtT_ref[0, c:c + 1, :]
        diff = mcol - grow
        D = D + diff * diff
    d_ref[...] = D

    iota = lax.broadcasted_iota(jnp.int32, (N, M), 1)
    rowmin = jnp.min(D, axis=1, keepdims=True)
    e1_sum = jnp.sum(jnp.sqrt(jnp.maximum(rowmin, 1e-9)))
    colmin = jnp.min(D, axis=0, keepdims=True)
    e2_sum = jnp.sum(jnp.sqrt(jnp.maximum(colmin, 1e-9)))
    idx0 = jnp.min(jnp.where(D == rowmin, iota, M), axis=1, keepdims=True)

    bits = lax.bitcast_convert_type(D, jnp.int32)
    lo = lax.bitcast_convert_type(rowmin, jnp.int32) - 1
    hi = lax.bitcast_convert_type(
        jnp.max(D, axis=1, keepdims=True), jnp.int32)

    def vcond(carry):
        lo, hi = carry
        return jnp.max(hi - lo) > 1

    def vbody(carry):
        lo, hi = carry
        mid = lo + (hi - lo) // 2
        cnt = jnp.sum((bits <= mid).astype(jnp.int32), axis=1, keepdims=True)
        ge = cnt >= (_K + 1)
        return jnp.where(ge, lo, mid), jnp.where(ge, mid, hi)

    lo, hi = lax.while_loop(vcond, vbody, (lo, hi))
    thr = hi

    eq = bits == thr
    n_eq = jnp.sum(eq.astype(jnp.int32))

    def no_ties():
        return jnp.full((N, 1), M - 1, jnp.int32)

    def with_ties():
        cnt_lt = jnp.sum((bits < thr).astype(jnp.int32), axis=1,
                         keepdims=True)
        need = (_K + 1) - cnt_lt
        lo2 = jnp.full((N, 1), -1, jnp.int32)
        hi2 = jnp.full((N, 1), M - 1, jnp.int32)

        def ibody(_, carry):
            lo, hi = carry
            mid = lo + (hi - lo) // 2
            cnt = jnp.sum((eq & (iota <= mid)).astype(jnp.int32), axis=1,
                          keepdims=True)
            ge = cnt >= need
            return jnp.where(ge, lo, mid), jnp.where(ge, mid, hi)

        lo2, hi2 = lax.fori_loop(0, 13, ibody, (lo2, hi2))
        return hi2

    istar = lax.cond(n_eq == N, no_ties, with_ties)

    thr_ref[...] = lax.bitcast_convert_type(thr, jnp.float32).reshape(1, 1, N)
    istar_ref[...] = istar.reshape(1, 1, N)
    idx0_ref[...] = idx0.reshape(1, 1, N)

    acc_ref[0] = acc_ref[0] + e1_sum
    acc_ref[1] = acc_ref[1] + e2_sum

    @pl.when(b == B - 1)
    def _finish():
        cd1 = (acc_ref[0] / (B * N) + acc_ref[1] / (B * M)) * 0.5 * 1000.0
        cd1_ref[...] = jnp.reshape(cd1, (1, 1))


def _tc_call(means, gtT, B, N, M):
    body = functools.partial(_tc_stage, B=B, N=N, M=M)
    return pl.pallas_call(
        body,
        grid=(B,),
        in_specs=[
            pl.BlockSpec((1, N, 3), lambda b: (b, 0, 0)),
            pl.BlockSpec((1, 3, M), lambda b: (b, 0, 0)),
        ],
        out_specs=[
            pl.BlockSpec((N, M), lambda b: (b, 0)),
            pl.BlockSpec((1, 1, N), lambda b: (b, 0, 0)),
            pl.BlockSpec((1, 1, N), lambda b: (b, 0, 0)),
            pl.BlockSpec((1, 1, N), lambda b: (b, 0, 0)),
            pl.BlockSpec((1, 1), lambda b: (0, 0)),
        ],
        out_shape=[
            jax.ShapeDtypeStruct((B * N, M), jnp.float32),
            jax.ShapeDtypeStruct((B, 1, N), jnp.float32),
            jax.ShapeDtypeStruct((B, 1, N), jnp.int32),
            jax.ShapeDtypeStruct((B, 1, N), jnp.int32),
            jax.ShapeDtypeStruct((1, 1), jnp.float32),
        ],
        scratch_shapes=[pltpu.SMEM((2,), jnp.float32)],
        compiler_params=pltpu.CompilerParams(
            dimension_semantics=("arbitrary",)),
    )(means, gtT)


def _sqrt16(x):
    b = lax.bitcast_convert_type(x, jnp.int32)
    y = lax.bitcast_convert_type((b >> 1) + 0x1FBD1DF5, jnp.float32)
    for _ in range(3):
        y = 0.5 * (y + x / y)
    return y


def _sc_call(dflat, thr, istar, idx0, spflat, gtflat, B, N, S, M):
    NC, NS, L = 2, 16, 16
    NW = NC * NS
    KPW = (B * N) // NW
    NCH = M // L
    KCH = (_K + L - 1) // L
    KPAD = KCH * L

    mesh = plsc.VectorSubcoreMesh(core_axis_name="c", subcore_axis_name="s")

    @functools.partial(
        pl.kernel, mesh=mesh,
        compiler_params=pltpu.CompilerParams(needs_layout_passes=False),
        out_type=jax.ShapeDtypeStruct((NW * L,), jnp.float32),
        scratch_types=[
            pltpu.VMEM((M,), jnp.float32),
            pltpu.VMEM((M,), jnp.float32),
            pltpu.VMEM((M,), jnp.float32),
            pltpu.VMEM((M,), jnp.float32),
            pltpu.VMEM((M,), jnp.float32),
            pltpu.VMEM((KPAD + L,), jnp.int32),
            pltpu.VMEM((KPAD,), jnp.float32),
            pltpu.VMEM((KPAD,), jnp.float32),
            pltpu.VMEM((KPAD,), jnp.float32),
            pltpu.VMEM((KPAD,), jnp.float32),
            pltpu.VMEM((KPW * S * 3,), jnp.float32),
            pltpu.VMEM((L,), jnp.float32),
            pltpu.VMEM((L,), jnp.int32),
            pltpu.VMEM((L,), jnp.int32),
            pltpu.VMEM((L,), jnp.float32),
            pltpu.SemaphoreType.DMA,
            pltpu.SemaphoreType.DMA,
        ],
    )
    def sc_kernel(d_hbm, thr_hbm, istar_hbm, idx0_hbm, sp_hbm, gt_hbm,
                  out_hbm, drow0, drow1, gx, gy, gz, cand, cgx, cgy, cgz,
                  minS, spv, thrv, istarv, idx0v, outbuf, sem0, sem1):
        wid = lax.axis_index("s") * NC + lax.axis_index("c")
        batch = (wid * KPW) // N
        lanes = lax.iota(jnp.int32, L)

        pltpu.sync_copy(gt_hbm.at[pl.ds((batch * 3 + 0) * M, M)], gx)
        pltpu.sync_copy(gt_hbm.at[pl.ds((batch * 3 + 1) * M, M)], gy)
        pltpu.sync_copy(gt_hbm.at[pl.ds((batch * 3 + 2) * M, M)], gz)
        pltpu.sync_copy(thr_hbm.at[pl.ds(wid * KPW, L)], thrv)
        pltpu.sync_copy(istar_hbm.at[pl.ds(wid * KPW, L)], istarv)
        pltpu.sync_copy(idx0_hbm.at[pl.ds(wid * KPW, L)], idx0v)
        pltpu.sync_copy(sp_hbm.at[pl.ds(wid * KPW * S * 3, KPW * S * 3)], spv)
        base = wid * KPW
        pltpu.async_copy(d_hbm.at[pl.ds(base * M, M)], drow0, sem0)

        for ch in range(KCH + 1):
            cand[pl.ds(ch * L, L)] = jnp.zeros((L,), jnp.int32)

        def _extract_f(vref, i):
            v = vref[...]
            return lax.reduce_max(jnp.where(lanes == i, v, -_BIG), axes=(0,))

        def _extract_i(vref, i):
            v = vref[...]
            return lax.reduce_max(
                jnp.where(lanes == i, v, -(2 ** 31 - 1)), axes=(0,))

        def per_kp(i, carry, drow):
            s1, s2 = carry
            thr_s = _extract_f(thrv, i)
            istar_s = _extract_i(istarv, i)
            idx0_s = _extract_i(idx0v, i)

            def compact(ch, cnt):
                dv = drow[pl.ds(ch * L, L)]
                idxv = lanes + ch * L
                m = (dv < thr_s) | ((dv == thr_s) & (idxv <= istar_s))
                m = m & (idxv != idx0_s)
                plsc.store_compressed(cand.at[pl.ds(cnt, L)], idxv, mask=m)
                pc = plsc.all_reduce_population_count(m)
                return cnt + pc[0]

            lax.fori_loop(0, NCH, compact, 0)

            def gather(ch, _):
                idxv = cand[pl.ds(ch * L, L)]
                cgx[pl.ds(ch * L, L)] = plsc.load_gather(gx, [idxv])
                cgy[pl.ds(ch * L, L)] = plsc.load_gather(gy, [idxv])
                cgz[pl.ds(ch * L, L)] = plsc.load_gather(gz, [idxv])
                minS[pl.ds(ch * L, L)] = jnp.full((L,), _BIG, jnp.float32)
                return 0

            lax.fori_loop(0, KCH, gather, 0)

            d1a = jnp.full((L,), 1e-9, jnp.float32)
            d1b = jnp.full((L,), 1e-9, jnp.float32)
            spo = i * (S * 3)
            for smp in range(S):
                svec = spv[pl.ds(spo + (smp // L) * L, L)]
                sx = svec[(smp % L)] * 1.0
                svec_y = spv[pl.ds(spo + S + (smp // L) * L, L)]
                sy = svec_y[(smp % L)] * 1.0
                svec_z = spv[pl.ds(spo + 2 * S + (smp // L) * L, L)]
                sz = svec_z[(smp % L)] * 1.0

                def per_chunk(ch, best, sx=sx, sy=sy, sz=sz):
                    valid = (lanes + ch * L) < _K
                    dx = sx - cgx[pl.ds(ch * L, L)]
                    dy = sy - cgy[pl.ds(ch * L, L)]
                    dz = sz - cgz[pl.ds(ch * L, L)]
                    dd = dx * dx + dy * dy + dz * dz
                    dd = jnp.where(valid, dd, _BIG)
                    mold = minS[pl.ds(ch * L, L)]
                    minS[pl.ds(ch * L, L)] = jnp.minimum(mold, dd)
                    return jnp.minimum(best, dd)

                best = lax.fori_loop(0, KCH, per_chunk,
                                     jnp.full((L,), _BIG, jnp.float32))
                bmin = lax.reduce_min(best, axes=(0,))
                if smp < L:
                    d1a = jnp.where(lanes == smp, bmin, d1a)
                else:
                    d1b = jnp.where(lanes == smp - L, bmin, d1b)
            s1 = s1 + lax.reduce_sum(
                _sqrt16(jnp.maximum(d1a, 1e-9)), axes=(0,))
            s1 = s1 + lax.reduce_sum(
                _sqrt16(jnp.maximum(d1b, 1e-9)), axes=(0,))

            def d2sum(ch, acc):
                valid = (lanes + ch * L) < _K
                mv = jnp.maximum(minS[pl.ds(ch * L, L)], 1e-9)
                rv = _sqrt16(mv)
                rv = jnp.where(valid, rv, 0.0)
                return acc + lax.reduce_sum(rv, axes=(0,))

            s2 = lax.fori_loop(0, KCH, d2sum, s2)
            return s1, s2

        def pair(t, carry):
            i0 = t * 2
            kp0 = base + i0
            pltpu.make_async_copy(
                d_hbm.at[pl.ds(kp0 * M, M)], drow0, sem0).wait()
            pltpu.async_copy(d_hbm.at[pl.ds((kp0 + 1) * M, M)], drow1, sem1)
            carry = per_kp(i0, carry, drow0)
            pltpu.make_async_copy(
                d_hbm.at[pl.ds((kp0 + 1) * M, M)], drow1, sem1).wait()

            @pl.when(t < (KPW // 2) - 1)
            def _prefetch():
                pltpu.async_copy(
                    d_hbm.at[pl.ds((kp0 + 2) * M, M)], drow0, sem0)

            return per_kp(i0 + 1, carry, drow1)

        s1, s2 = lax.fori_loop(0, KPW // 2, pair, (0.0, 0.0))

        outbuf[...] = jnp.where(lanes == 0, s1,
                                jnp.where(lanes == 1, s2, 0.0))
        pltpu.sync_copy(outbuf, out_hbm.at[pl.ds(wid * L, L)])

    return sc_kernel(dflat, thr, istar, idx0, spflat, gtflat)


@jax.jit
def kernel(means, sample_points, gt):
    B, N, S, _ = sample_points.shape
    M = gt.shape[1]
    gtT = gt.transpose(0, 2, 1)

    dmat, thr, istar, idx0, cd1 = _tc_call(means, gtT, B, N, M)

    part = _sc_call(dmat.reshape(B * N * M), thr.reshape(B * N),
                    istar.reshape(B * N), idx0.reshape(B * N),
                    sample_points.transpose(0, 1, 3, 2).reshape(B * N * S * 3),
                    gtT.reshape(B * 3 * M), B, N, S, M)
    part = part.reshape(32, 16)
    S1 = jnp.sum(part[:, 0])
    S2 = jnp.sum(part[:, 1])
    cd2 = (S1 / (B * S) + S2 / (B * _K)) * 0.5 * 1000.0
    cd1s = cd1[0, 0]
    return (cd2, cd1s, cd2)


# speedup vs baseline: 1.2513x; 1.1291x over previous
import functools

import jax
import jax.numpy as jnp
from jax import lax
from jax.experimental import pallas as pl
from jax.experimental.pallas import tpu as pltpu
from jax.experimental.pallas import tpu_sc as plsc

_K = 200
_BIG = 1e30


def _tc_stage(means_ref, gtT_ref, d_ref, thr_ref, istar_ref, idx0_ref,
              cd1_ref, acc_ref, *, B, N, M):
    b = pl.program_id(0)

    @pl.when(b == 0)
    def _init():
        acc_ref[0] = 0.0
        acc_ref[1] = 0.0

    D = jnp.zeros((N, M), jnp.float32)
    for c in range(3):
        mcol = means_ref[0, :, c:c + 1]
        grow = gtT_ref[0, c:c + 1, :]
        diff = mcol - grow
        D = D + diff * diff
    d_ref[...] = D

    iota = lax.broadcasted_iota(jnp.int32, (N, M), 1)
    rowmin = jnp.min(D, axis=1, keepdims=True)
    e1_sum = jnp.sum(jnp.sqrt(jnp.maximum(rowmin, 1e-9)))
    colmin = jnp.min(D, axis=0, keepdims=True)
    e2_sum = jnp.sum(jnp.sqrt(jnp.maximum(colmin, 1e-9)))
    idx0 = jnp.min(jnp.where(D == rowmin, iota, M), axis=1, keepdims=True)

    bits = lax.bitcast_convert_type(D, jnp.int32)
    lo = lax.bitcast_convert_type(rowmin, jnp.int32) - 1
    hi = lax.bitcast_convert_type(
        jnp.max(D, axis=1, keepdims=True), jnp.int32)

    def vcond(carry):
        lo, hi = carry
        return jnp.max(hi - lo) > 1

    def vbody(carry):
        lo, hi = carry
        mid = lo + (hi - lo) // 2
        cnt = jnp.sum((bits <= mid).astype(jnp.int32), axis=1, keepdims=True)
        ge = cnt >= (_K + 1)
        return jnp.where(ge, lo, mid), jnp.where(ge, mid, hi)

    lo, hi = lax.while_loop(vcond, vbody, (lo, hi))
    thr = hi

    eq = bits == thr
    n_eq = jnp.sum(eq.astype(jnp.int32))

    def no_ties():
        return jnp.full((N, 1), M - 1, jnp.int32)

    def with_ties():
        cnt_lt = jnp.sum((bits < thr).astype(jnp.int32), axis=1,
                         keepdims=True)
        need = (_K + 1) - cnt_lt
        lo2 = jnp.full((N, 1), -1, jnp.int32)
        hi2 = jnp.full((N, 1), M - 1, jnp.int32)

        def ibody(_, carry):
            lo, hi = carry
            mid = lo + (hi - lo) // 2
            cnt = jnp.sum((eq & (iota <= mid)).astype(jnp.int32), axis=1,
                          keepdims=True)
            ge = cnt >= need
            return jnp.where(ge, lo, mid), jnp.where(ge, mid, hi)

        lo2, hi2 = lax.fori_loop(0, 13, ibody, (lo2, hi2))
        return hi2

    istar = lax.cond(n_eq == N, no_ties, with_ties)

    thr_ref[...] = lax.bitcast_convert_type(thr, jnp.float32).reshape(1, 1, N)
    istar_ref[...] = istar.reshape(1, 1, N)
    idx0_ref[...] = idx0.reshape(1, 1, N)

    acc_ref[0] = acc_ref[0] + e1_sum
    acc_ref[1] = acc_ref[1] + e2_sum

    @pl.when(b == B - 1)
    def _finish():
        cd1 = (acc_ref[0] / (B * N) + acc_ref[1] / (B * M)) * 0.5 * 1000.0
        cd1_ref[...] = jnp.reshape(cd1, (1, 1))


def _tc_call(means, gtT, B, N, M):
    body = functools.partial(_tc_stage, B=B, N=N, M=M)
    return pl.pallas_call(
        body,
        grid=(B,),
        in_specs=[
            pl.BlockSpec((1, N, 3), lambda b: (b, 0, 0)),
            pl.BlockSpec((1, 3, M), lambda b: (b, 0, 0)),
        ],
        out_specs=[
            pl.BlockSpec((N, M), lambda b: (b, 0)),
            pl.BlockSpec((1, 1, N), lambda b: (b, 0, 0)),
            pl.BlockSpec((1, 1, N), lambda b: (b, 0, 0)),
            pl.BlockSpec((1, 1, N), lambda b: (b, 0, 0)),
            pl.BlockSpec((1, 1), lambda b: (0, 0)),
        ],
        out_shape=[
            jax.ShapeDtypeStruct((B * N, M), jnp.float32),
            jax.ShapeDtypeStruct((B, 1, N), jnp.float32),
            jax.ShapeDtypeStruct((B, 1, N), jnp.int32),
            jax.ShapeDtypeStruct((B, 1, N), jnp.int32),
            jax.ShapeDtypeStruct((1, 1), jnp.float32),
        ],
        scratch_shapes=[pltpu.SMEM((2,), jnp.float32)],
        compiler_params=pltpu.CompilerParams(
            dimension_semantics=("arbitrary",)),
    )(means, gtT)


def _sqrt16(x):
    b = lax.bitcast_convert_type(x, jnp.int32)
    y = lax.bitcast_convert_type((b >> 1) + 0x1FBD1DF5, jnp.float32)
    for _ in range(3):
        y = 0.5 * (y + x / y)
    return y


def _sc_call(dflat, thr, istar, idx0, spflat, gtflat, B, N, S, M):
    NC, NS, L = 2, 16, 16
    NW = NC * NS
    KPW = (B * N) // NW
    NCH = M // L
    KCH = (_K + L - 1) // L
    KPAD = KCH * L

    mesh = plsc.VectorSubcoreMesh(core_axis_name="c", subcore_axis_name="s")

    @functools.partial(
        pl.kernel, mesh=mesh,
        compiler_params=pltpu.CompilerParams(needs_layout_passes=False),
        out_type=jax.ShapeDtypeStruct((NW * L,), jnp.float32),
        scratch_types=[
            pltpu.VMEM((M,), jnp.float32),
            pltpu.VMEM((M,), jnp.float32),
            pltpu.VMEM((M,), jnp.float32),
            pltpu.VMEM((M,), jnp.float32),
            pltpu.VMEM((M,), jnp.float32),
            pltpu.VMEM((KPAD + L,), jnp.int32),
            pltpu.VMEM((KPAD,), jnp.float32),
            pltpu.VMEM((KPAD,), jnp.float32),
            pltpu.VMEM((KPAD,), jnp.float32),
            pltpu.VMEM((KPAD,), jnp.float32),
            pltpu.VMEM((KPW * S * 3,), jnp.float32),
            pltpu.VMEM((L,), jnp.float32),
            pltpu.VMEM((L,), jnp.int32),
            pltpu.VMEM((L,), jnp.int32),
            pltpu.VMEM((L,), jnp.float32),
            pltpu.SemaphoreType.DMA,
            pltpu.SemaphoreType.DMA,
        ],
    )
    def sc_kernel(d_hbm, thr_hbm, istar_hbm, idx0_hbm, sp_hbm, gt_hbm,
                  out_hbm, drow0, drow1, gx, gy, gz, cand, cgx, cgy, cgz,
                  minS, spv, thrv, istarv, idx0v, outbuf, sem0, sem1):
        wid = lax.axis_index("s") * NC + lax.axis_index("c")
        batch = (wid * KPW) // N
        lanes = lax.iota(jnp.int32, L)

        pltpu.sync_copy(gt_hbm.at[pl.ds((batch * 3 + 0) * M, M)], gx)
        pltpu.sync_copy(gt_hbm.at[pl.ds((batch * 3 + 1) * M, M)], gy)
        pltpu.sync_copy(gt_hbm.at[pl.ds((batch * 3 + 2) * M, M)], gz)
        pltpu.sync_copy(thr_hbm.at[pl.ds(wid * KPW, L)], thrv)
        pltpu.sync_copy(istar_hbm.at[pl.ds(wid * KPW, L)], istarv)
        pltpu.sync_copy(idx0_hbm.at[pl.ds(wid * KPW, L)], idx0v)
        pltpu.sync_copy(sp_hbm.at[pl.ds(wid * KPW * S * 3, KPW * S * 3)], spv)
        base = wid * KPW
        pltpu.async_copy(d_hbm.at[pl.ds(base * M, M)], drow0, sem0)

        for ch in range(KCH + 1):
            cand[pl.ds(ch * L, L)] = jnp.zeros((L,), jnp.int32)

        def _extract_f(vref, i):
            v = vref[...]
            return lax.reduce_max(jnp.where(lanes == i, v, -_BIG), axes=(0,))

        def _extract_i(vref, i):
            v = vref[...]
            return lax.reduce_max(
                jnp.where(lanes == i, v, -(2 ** 31 - 1)), axes=(0,))

        def per_kp(i, carry, drow):
            s1, s2 = carry
            thr_s = _extract_f(thrv, i)
            istar_s = _extract_i(istarv, i)
            idx0_s = _extract_i(idx0v, i)

            def compact(ch, cnt):
                dv = drow[pl.ds(ch * L, L)]
                idxv = lanes + ch * L
                m = (dv < thr_s) | ((dv == thr_s) & (idxv <= istar_s))
                m = m & (idxv != idx0_s)
                plsc.store_compressed(cand.at[pl.ds(cnt, L)], idxv, mask=m)
                pc = plsc.all_reduce_population_count(m)
                return cnt + pc[0]

            lax.fori_loop(0, NCH, compact, 0)

            def gather(ch, _):
                idxv = cand[pl.ds(ch * L, L)]
                cgx[pl.ds(ch * L, L)] = plsc.load_gather(gx, [idxv])
                cgy[pl.ds(ch * L, L)] = plsc.load_gather(gy, [idxv])
                cgz[pl.ds(ch * L, L)] = plsc.load_gather(gz, [idxv])
                minS[pl.ds(ch * L, L)] = jnp.full((L,), _BIG, jnp.float32)
                return 0

            lax.fori_loop(0, KCH, gather, 0)
            tail = _K - (KCH - 1) * L
            tmask = lanes >= tail
            tslice = pl.ds((KCH - 1) * L, L)
            cgx[tslice] = jnp.where(tmask, 1e15, cgx[tslice])

            d1a = jnp.full((L,), 1e-9, jnp.float32)
            d1b = jnp.full((L,), 1e-9, jnp.float32)
            spo = i * (S * 3)
            for smp in range(S):
                svec = spv[pl.ds(spo + (smp // L) * L, L)]
                sx = svec[(smp % L)] * 1.0
                svec_y = spv[pl.ds(spo + S + (smp // L) * L, L)]
                sy = svec_y[(smp % L)] * 1.0
                svec_z = spv[pl.ds(spo + 2 * S + (smp // L) * L, L)]
                sz = svec_z[(smp % L)] * 1.0

                def per_chunk(ch, best, sx=sx, sy=sy, sz=sz):
                    dx = sx - cgx[pl.ds(ch * L, L)]
                    dy = sy - cgy[pl.ds(ch * L, L)]
                    dz = sz - cgz[pl.ds(ch * L, L)]
                    dd = dx * dx + dy * dy + dz * dz
                    mold = minS[pl.ds(ch * L, L)]
                    minS[pl.ds(ch * L, L)] = jnp.minimum(mold, dd)
                    return jnp.minimum(best, dd)

                best = lax.fori_loop(0, KCH, per_chunk,
                                     jnp.full((L,), _BIG, jnp.float32))
                bmin = lax.reduce_min(best, axes=(0,))
                if smp < L:
                    d1a = jnp.where(lanes == smp, bmin, d1a)
                else:
                    d1b = jnp.where(lanes == smp - L, bmin, d1b)
            s1 = s1 + lax.reduce_sum(
                _sqrt16(jnp.maximum(d1a, 1e-9)), axes=(0,))
            s1 = s1 + lax.reduce_sum(
                _sqrt16(jnp.maximum(d1b, 1e-9)), axes=(0,))

            def d2sum(ch, acc):
                valid = (lanes + ch * L) < _K
                mv = jnp.maximum(minS[pl.ds(ch * L, L)], 1e-9)
                rv = _sqrt16(mv)
                rv = jnp.where(valid, rv, 0.0)
                return acc + lax.reduce_sum(rv, axes=(0,))

            s2 = lax.fori_loop(0, KCH, d2sum, s2)
            return s1, s2

        def pair(t, carry):
            i0 = t * 2
            kp0 = base + i0
            pltpu.make_async_copy(
                d_hbm.at[pl.ds(kp0 * M, M)], drow0, sem0).wait()
            pltpu.async_copy(d_hbm.at[pl.ds((kp0 + 1) * M, M)], drow1, sem1)
            carry = per_kp(i0, carry, drow0)
            pltpu.make_async_copy(
                d_hbm.at[pl.ds((kp0 + 1) * M, M)], drow1, sem1).wait()

            @pl.when(t < (KPW // 2) - 1)
            def _prefetch():
                pltpu.async_copy(
                    d_hbm.at[pl.ds((kp0 + 2) * M, M)], drow0, sem0)

            return per_kp(i0 + 1, carry, drow1)

        s1, s2 = lax.fori_loop(0, KPW // 2, pair, (0.0, 0.0))

        outbuf[...] = jnp.where(lanes == 0, s1,
                                jnp.where(lanes == 1, s2, 0.0))
        pltpu.sync_copy(outbuf, out_hbm.at[pl.ds(wid * L, L)])

    return sc_kernel(dflat, thr, istar, idx0, spflat, gtflat)


@jax.jit
def kernel(means, sample_points, gt):
    B, N, S, _ = sample_points.shape
    M = gt.shape[1]
    gtT = gt.transpose(0, 2, 1)

    dmat, thr, istar, idx0, cd1 = _tc_call(means, gtT, B, N, M)

    part = _sc_call(dmat.reshape(B * N * M), thr.reshape(B * N),
                    istar.reshape(B * N), idx0.reshape(B * N),
                    sample_points.transpose(0, 1, 3, 2).reshape(B * N * S * 3),
                    gtT.reshape(B * 3 * M), B, N, S, M)
    part = part.reshape(32, 16)
    S1 = jnp.sum(part[:, 0])
    S2 = jnp.sum(part[:, 1])
    cd2 = (S1 / (B * S) + S2 / (B * _K)) * 0.5 * 1000.0
    cd1s = cd1[0, 0]
    return (cd2, cd1s, cd2)
